# baseline (speedup 1.0000x reference)
import functools
import math

import jax
import jax.numpy as jnp
from jax import lax
from jax.experimental import pallas as pl
from jax.experimental.pallas import tpu as pltpu
from jax.experimental.pallas import tpu_sc as plsc

B = 16384
D = 128
K = 16
NW = 32
RW = B // NW

_LN2 = math.log(2.0)


def _vector_log(s):
    e2 = jnp.where(s >= 4.0, 2.0, 0.0)
    s2 = jnp.where(s >= 4.0, s * 0.25, s)
    e1 = jnp.where(s2 >= 2.0, 1.0, 0.0)
    s1 = jnp.where(s2 >= 2.0, s2 * 0.5, s2)
    w = (s1 - 1.0) / (s1 + 1.0)
    w2 = w * w
    poly = 1.0 + w2 * (1.0 / 3.0 + w2 * (0.2 + w2 * (1.0 / 7.0)))
    return 2.0 * w * poly + (e2 + e1) * _LN2


def _make_sc_kernel():
    mesh = plsc.VectorSubcoreMesh(core_axis_name="c", subcore_axis_name="s")

    @functools.partial(
        pl.kernel,
        mesh=mesh,
        out_type=jax.ShapeDtypeStruct((B,), jnp.float32),
        compiler_params=pltpu.CompilerParams(needs_layout_passes=False),
        scratch_types=[
            pltpu.VMEM((RW, D), jnp.float32),
            pltpu.VMEM((D, K), jnp.float32),
            pltpu.VMEM((D, K), jnp.float32),
            pltpu.VMEM((K,), jnp.float32),
            pltpu.VMEM((RW,), jnp.float32),
        ],
    )
    def k(p1_hbm, p2_hbm, c_hbm, value_hbm, out_hbm, v_v, p1_v, p2_v, c_v, o_v):
        wid = lax.axis_index("s") * 2 + lax.axis_index("c")
        base = wid * RW
        pltpu.sync_copy(p1_hbm, p1_v)
        pltpu.sync_copy(p2_hbm, p2_v)
        pltpu.sync_copy(c_hbm, c_v)
        pltpu.sync_copy(value_hbm.at[pl.ds(base, RW)], v_v)
        cvec = c_v[...]
        lanes = lax.iota(jnp.int32, K)

        def rowgroup(rg, carry):
            mxv = jnp.zeros((K,), jnp.float32)
            sv = jnp.zeros((K,), jnp.float32)
            for j in range(16):
                r = rg * 16 + j
                acc = cvec
                for cchunk in range(D // 16):
                    vv = v_v[r, pl.ds(cchunk * 16, 16)]
                    for t in range(16):
                        d = cchunk * 16 + t
                        vb = lax.broadcast_in_dim(vv[t], (K,), ())
                        acc = acc + (p1_v[d, :] * vb + p2_v[d, :]) * vb
                mxb = lax.broadcast_in_dim(plsc.cummax(acc)[15], (K,), ())
                e = jnp.exp(acc - mxb)
                sb = lax.broadcast_in_dim(plsc.cumsum(e)[15], (K,), ())
                lane = lanes == j
                mxv = jnp.where(lane, mxb, mxv)
                sv = jnp.where(lane, sb, sv)
            o_v[pl.ds(rg * 16, 16)] = mxv + _vector_log(sv)
            return carry

        lax.fori_loop(0, RW // 16, rowgroup, 0)
        pltpu.sync_copy(o_v, out_hbm.at[pl.ds(base, RW)])

    return k


_sc_kernel = _make_sc_kernel()


def kernel(value, means, log_stds, log_weights):
    inv_var = jnp.exp(-2.0 * log_stds)
    p1 = jnp.transpose(-0.5 * inv_var)
    p2 = jnp.transpose(means * inv_var)
    c = (-0.5 * jnp.sum(means * means * inv_var, axis=1)
         - jnp.sum(log_stds, axis=1)
         - 0.5 * D * math.log(2.0 * math.pi)
         + log_weights)
    return _sc_kernel(p1, p2, c, value)


# speedup vs baseline: 57.6360x; 57.6360x over previous
import functools
import math

import jax
import jax.numpy as jnp
from jax.experimental import pallas as pl
from jax.experimental.pallas import tpu as pltpu

B = 16384
D = 128
K = 16
BB = 8192


def _mixture_kernel(value_ref, means_ref, log_stds_ref, log_weights_ref, out_ref):
    v = value_ref[...]
    log_stds = log_stds_ref[...]
    means = means_ref[...]
    lw = log_weights_ref[...]

    inv_var = jnp.exp(-2.0 * log_stds)
    a = means * inv_var
    c = (-0.5 * jnp.sum(means * a, axis=1)
         - jnp.sum(log_stds, axis=1)
         - 0.5 * D * math.log(2.0 * math.pi)
         + lw[0])

    q = jax.lax.dot_general(inv_var, v * v, (((1,), (1,)), ((), ())),
                            preferred_element_type=jnp.float32)
    l = jax.lax.dot_general(a, v, (((1,), (1,)), ((), ())),
                            preferred_element_type=jnp.float32)
    comp = (-0.5 * q + l) + c[:, None]

    m = jnp.max(comp, axis=0, keepdims=True)
    s = jnp.sum(jnp.exp(comp - m), axis=0, keepdims=True)
    out_ref[...] = (m + jnp.log(s))[0]


@functools.partial(jax.jit, static_argnames=())
def kernel(value, means, log_stds, log_weights):
    lw2 = log_weights.reshape(1, K)
    out = pl.pallas_call(
        _mixture_kernel,
        grid=(B // BB,),
        in_specs=[
            pl.BlockSpec((BB, D), lambda i: (i, 0)),
            pl.BlockSpec((K, D), lambda i: (0, 0)),
            pl.BlockSpec((K, D), lambda i: (0, 0)),
            pl.BlockSpec((1, K), lambda i: (0, 0)),
        ],
        out_specs=pl.BlockSpec((BB,), lambda i: (i,)),
        out_shape=jax.ShapeDtypeStruct((B,), jnp.float32),
        compiler_params=pltpu.CompilerParams(
            dimension_semantics=("parallel",),
        ),
    )(value, means, log_stds, lw2)
    return out
